# baseline (speedup 1.0000x reference)
import functools

import numpy as np

import jax
import jax.numpy as jnp
from jax import lax
from jax.experimental import pallas as pl
from jax.experimental.pallas import tpu as pltpu

_EPS = 1e-5
_HI = lax.Precision.HIGHEST


def _fold_matrices(C, lanes=128):
    f = (np.arange(lanes)[:, None] % C == np.arange(C)[None, :]).astype(np.float32)
    return jnp.asarray(f), jnp.asarray(f.T)


def _norm_kernel(x_ref, w_ref, b_ref, f_ref, ft_ref, y_ref, mean_ref, std_ref,
                 *, inv_t):
    x = x_ref[...]
    s = jnp.sum(x, axis=1)
    sq = jnp.sum(x * x, axis=1)
    f = f_ref[...]
    mean = jnp.dot(s, f, precision=_HI,
                   preferred_element_type=jnp.float32) * inv_t
    msq = jnp.dot(sq, f, precision=_HI,
                  preferred_element_type=jnp.float32) * inv_t
    var = msq - mean * mean
    std = jnp.sqrt(var + _EPS)
    scale = w_ref[...] / std
    shift = b_ref[...] - mean * scale
    ft = ft_ref[...]
    scale_l = jnp.dot(scale, ft, precision=_HI,
                      preferred_element_type=jnp.float32)
    shift_l = jnp.dot(shift, ft, precision=_HI,
                      preferred_element_type=jnp.float32)
    y_ref[...] = x * scale_l[:, None, :] + shift_l[:, None, :]
    mean_ref[...] = mean
    std_ref[...] = std


def _run_chunk(xg, w2, b2, f, ft, inv_t):
    Bc, G, lanes = xg.shape
    C = w2.shape[1]
    bb = 128
    while Bc % bb != 0:
        bb //= 2
    grid = (Bc // bb,)
    body = functools.partial(_norm_kernel, inv_t=inv_t)
    return pl.pallas_call(
        body,
        out_shape=(jax.ShapeDtypeStruct((Bc, G, lanes), xg.dtype),
                   jax.ShapeDtypeStruct((Bc, C), jnp.float32),
                   jax.ShapeDtypeStruct((Bc, C), jnp.float32)),
        grid=grid,
        in_specs=[
            pl.BlockSpec((bb, G, lanes), lambda i: (i, 0, 0)),
            pl.BlockSpec((1, C), lambda i: (0, 0)),
            pl.BlockSpec((1, C), lambda i: (0, 0)),
            pl.BlockSpec((lanes, C), lambda i: (0, 0)),
            pl.BlockSpec((C, lanes), lambda i: (0, 0)),
        ],
        out_specs=[
            pl.BlockSpec((bb, G, lanes), lambda i: (i, 0, 0)),
            pl.BlockSpec((bb, C), lambda i: (i, 0)),
            pl.BlockSpec((bb, C), lambda i: (i, 0)),
        ],
        compiler_params=pltpu.CompilerParams(
            dimension_semantics=("parallel",),
            vmem_limit_bytes=48 << 20,
        ),
    )(xg, w2, b2, f, ft)


def kernel(x, affine_weight, affine_bias):
    B, T, C = x.shape
    L = T * C
    lanes = 128
    assert L % lanes == 0 and lanes % C == 0
    G = L // lanes
    inv_t = float(1.0 / T)

    f, ft = _fold_matrices(C, lanes)
    w2 = affine_weight.astype(jnp.float32).reshape(1, C)
    b2 = affine_bias.astype(jnp.float32).reshape(1, C)

    K = 4
    while B % K != 0:
        K //= 2
    Bc = B // K

    ys, means, stds = [], [], []
    for k in range(K):
        xc = lax.slice_in_dim(x, k * Bc, (k + 1) * Bc, axis=0)
        xg = xc.reshape(Bc, G, lanes)
        y_k, mean_k, std_k = _run_chunk(xg, w2, b2, f, ft, inv_t)
        ys.append(y_k.reshape(Bc, T, C))
        means.append(mean_k)
        stds.append(std_k)

    y = jnp.concatenate(ys, axis=0)
    mean = jnp.concatenate(means, axis=0).reshape(B, 1, C)
    std = jnp.concatenate(stds, axis=0).reshape(B, 1, C)
    return y, mean, std


# speedup vs baseline: 1.4649x; 1.4649x over previous
import functools

import numpy as np

import jax
import jax.numpy as jnp
from jax import lax
from jax.experimental import pallas as pl
from jax.experimental.pallas import tpu as pltpu

_EPS = 1e-5
_HI = lax.Precision.HIGHEST


def _fold_matrices(C, lanes=128):
    f = (np.arange(lanes)[:, None] % C == np.arange(C)[None, :]).astype(np.float32)
    return jnp.asarray(f), jnp.asarray(f.T)


def _norm_kernel(x_ref, w_ref, b_ref, f_ref, ft_ref, y_ref, mean_ref, std_ref,
                 *, inv_t):
    x = x_ref[...]
    s = jnp.sum(x, axis=1)
    sq = jnp.sum(x * x, axis=1)
    f = f_ref[...]
    mean = jnp.dot(s, f, precision=_HI,
                   preferred_element_type=jnp.float32) * inv_t
    msq = jnp.dot(sq, f, precision=_HI,
                  preferred_element_type=jnp.float32) * inv_t
    var = msq - mean * mean
    std = jnp.sqrt(var + _EPS)
    scale = w_ref[...] / std
    shift = b_ref[...] - mean * scale
    ft = ft_ref[...]
    scale_l = jnp.dot(scale, ft, precision=_HI,
                      preferred_element_type=jnp.float32)
    shift_l = jnp.dot(shift, ft, precision=_HI,
                      preferred_element_type=jnp.float32)
    y_ref[...] = x * scale_l[:, None, :] + shift_l[:, None, :]
    mean_ref[...] = mean
    std_ref[...] = std


def _run_chunk(xg, w2, b2, f, ft, inv_t):
    Bc, G, lanes = xg.shape
    C = w2.shape[1]
    bb = 256
    while Bc % bb != 0:
        bb //= 2
    grid = (Bc // bb,)
    body = functools.partial(_norm_kernel, inv_t=inv_t)
    return pl.pallas_call(
        body,
        out_shape=(jax.ShapeDtypeStruct((Bc, G, lanes), xg.dtype),
                   jax.ShapeDtypeStruct((Bc, C), jnp.float32),
                   jax.ShapeDtypeStruct((Bc, C), jnp.float32)),
        grid=grid,
        in_specs=[
            pl.BlockSpec((bb, G, lanes), lambda i: (i, 0, 0)),
            pl.BlockSpec((1, C), lambda i: (0, 0)),
            pl.BlockSpec((1, C), lambda i: (0, 0)),
            pl.BlockSpec((lanes, C), lambda i: (0, 0)),
            pl.BlockSpec((C, lanes), lambda i: (0, 0)),
        ],
        out_specs=[
            pl.BlockSpec((bb, G, lanes), lambda i: (i, 0, 0)),
            pl.BlockSpec((bb, C), lambda i: (i, 0)),
            pl.BlockSpec((bb, C), lambda i: (i, 0)),
        ],
        compiler_params=pltpu.CompilerParams(
            dimension_semantics=("parallel",),
            vmem_limit_bytes=48 << 20,
        ),
    )(xg, w2, b2, f, ft)


def kernel(x, affine_weight, affine_bias):
    B, T, C = x.shape
    L = T * C
    lanes = 128
    assert L % lanes == 0 and lanes % C == 0
    G = L // lanes
    inv_t = float(1.0 / T)

    f, ft = _fold_matrices(C, lanes)
    w2 = affine_weight.astype(jnp.float32).reshape(1, C)
    b2 = affine_bias.astype(jnp.float32).reshape(1, C)

    K = 1
    while B % K != 0:
        K //= 2
    Bc = B // K

    ys, means, stds = [], [], []
    for k in range(K):
        xc = lax.slice_in_dim(x, k * Bc, (k + 1) * Bc, axis=0)
        xg = xc.reshape(Bc, G, lanes)
        y_k, mean_k, std_k = _run_chunk(xg, w2, b2, f, ft, inv_t)
        ys.append(y_k.reshape(Bc, T, C))
        means.append(mean_k)
        stds.append(std_k)

    y = jnp.concatenate(ys, axis=0)
    mean = jnp.concatenate(means, axis=0).reshape(B, 1, C)
    std = jnp.concatenate(stds, axis=0).reshape(B, 1, C)
    return y, mean, std
